# baseline (speedup 1.0000x reference)
import functools

import jax
import jax.numpy as jnp
from jax import lax
from jax.experimental import pallas as pl
from jax.experimental.pallas import tpu as pltpu
from jax.experimental.pallas import tpu_sc as plsc

_B = 16384
_D = 32
_NC = 2
_NS = 16
_NW = _NC * _NS
_BPW = _B // _NW

_mesh = plsc.VectorSubcoreMesh(core_axis_name="c", subcore_axis_name="s")


@functools.partial(
    pl.kernel,
    mesh=_mesh,
    compiler_params=pltpu.CompilerParams(use_tc_tiling_on_sc=False),
    out_type=[
        jax.ShapeDtypeStruct((_B, _D), jnp.float32),
        jax.ShapeDtypeStruct((_B, _D), jnp.float32),
    ],
    scratch_types=[
        pltpu.VMEM((_BPW,), jnp.int32),
        pltpu.VMEM((_BPW,), jnp.int32),
        pltpu.VMEM((_BPW, _D), jnp.float32),
        pltpu.VMEM((_BPW, _D), jnp.float32),
        pltpu.SemaphoreType.DMA,
        pltpu.SemaphoreType.DMA,
    ],
)
def _sc_gather2(user_table, item_table, user_id, item_id, u_out, i_out,
                uidx_v, iidx_v, urows_v, irows_v, sem_u, sem_i):
    wid = lax.axis_index("s") * _NC + lax.axis_index("c")
    base = wid * _BPW
    pltpu.sync_copy(user_id.at[pl.ds(base, _BPW)], uidx_v)
    pltpu.sync_copy(item_id.at[pl.ds(base, _BPW)], iidx_v)
    cu = pltpu.async_copy(user_table.at[uidx_v], urows_v, sem_u)
    ci = pltpu.async_copy(item_table.at[iidx_v], irows_v, sem_i)
    cu.wait()
    pltpu.sync_copy(urows_v, u_out.at[pl.ds(base, _BPW)])
    ci.wait()
    pltpu.sync_copy(irows_v, i_out.at[pl.ds(base, _BPW)])


_BLK = 2048


def _mlp_body(ue_ref, ie_ref, w1_ref, b1_ref, w2_ref, b2_ref, w3_ref, b3_ref,
              wo_ref, bo_ref, out_ref):
    dn = (((1,), (1,)), ((), ()))
    w1 = w1_ref[...]
    h = lax.dot_general(ue_ref[...], w1[:, :_D], dn)
    h = h + lax.dot_general(ie_ref[...], w1[:, _D:], dn)
    h = jnp.maximum(h + b1_ref[...], 0.0)
    h = jnp.maximum(lax.dot_general(h, w2_ref[...], dn) + b2_ref[...], 0.0)
    h = jnp.maximum(lax.dot_general(h, w3_ref[...], dn) + b3_ref[...], 0.0)
    y = lax.dot_general(h, wo_ref[...], dn)
    out_ref[...] = y[:, :1] + bo_ref[0]


def _tc_mlp(ue, ie, W1, b1, W2, b2, W3, b3, Wo, bo):
    grid = (_B // _BLK,)
    row_spec = pl.BlockSpec((_BLK, _D), lambda i: (i, 0))

    def _full(a):
        return pl.BlockSpec(a.shape, lambda i: tuple(0 for _ in a.shape))

    return pl.pallas_call(
        _mlp_body,
        grid=grid,
        in_specs=[row_spec, row_spec, _full(W1), _full(b1), _full(W2),
                  _full(b2), _full(W3), _full(b3), _full(Wo),
                  pl.BlockSpec(memory_space=pltpu.SMEM)],
        out_specs=pl.BlockSpec((_BLK, 1), lambda i: (i, 0)),
        out_shape=jax.ShapeDtypeStruct((_B, 1), jnp.float32),
    )(ue, ie, W1, b1, W2, b2, W3, b3, Wo, bo)


def kernel(user_id, item_id, user_table, item_table, W1, b1, W2, b2, W3, b3,
           Wo, bo):
    ue, ie = _sc_gather2(user_table, item_table,
                         user_id.astype(jnp.int32), item_id.astype(jnp.int32))
    out = _tc_mlp(ue, ie, W1, b1.reshape(1, -1), W2, b2.reshape(1, -1),
                  W3, b3.reshape(1, -1), jnp.pad(Wo, ((0, 7), (0, 0))), bo)
    return out.reshape(_B)
